# Initial kernel scaffold; baseline (speedup 1.0000x reference)
import jax
import jax.numpy as jnp
from jax.experimental import pallas as pl


def kernel(x, W1, b1, W2, b2, W3, b3, Wl, bl, Wm1, bm1, Wm2, bm2, Wh, bh):
    raise NotImplementedError("write your pallas kernel here")


# speedup vs baseline: 8.9311x; 8.9311x over previous
import functools

import jax
import jax.numpy as jnp
from jax import lax
from jax.experimental import pallas as pl
from jax.experimental.pallas import tpu as pltpu
from jax.experimental.pallas import tpu_sc as plsc

_N = 4096
_K = 32
_OUT = 64
_NEG = float(jnp.finfo(jnp.float32).min)


_TOPK_R = 256


def _dot(a, b):
    return lax.dot_general(a, b, (((1,), (0,)), ((), ())),
                           preferred_element_type=jnp.float32)


def _topk_body(x_ref, xT_ref, wu_ref, wa_ref, b_ref,
               idx_ref, u_ref, a_ref, s_ref):
    R = _TOPK_R
    xb = x_ref[...]
    xt = xT_ref[...]
    xy = _dot(xb, xt)
    sq = jnp.sum(xt * xt, axis=0, keepdims=True)
    s_ref[...] = 2.0 * xy - sq
    u_ref[...] = _dot(xb, wu_ref[...])
    a_ref[...] = _dot(xb, wa_ref[...]) + b_ref[...]
    iota = lax.broadcasted_iota(jnp.int32, (R, _N), 1)
    lanek = lax.broadcasted_iota(jnp.int32, (R, _K), 1)

    def step(r, idx_acc):
        sw = s_ref[...]
        v = jnp.max(sw, axis=1, keepdims=True)
        jstar = jnp.min(jnp.where(sw == v, iota, _N), axis=1, keepdims=True)
        s_ref[...] = jnp.where(iota == jstar, _NEG, sw)
        return jnp.where(lanek == r, jstar, idx_acc)

    idx_ref[...] = lax.fori_loop(0, _K, step, jnp.zeros((R, _K), jnp.int32))


def _edge_topk(x, Wu, Wa, b2):
    C = x.shape[1]
    R = _TOPK_R
    G = _N // R
    return pl.pallas_call(
        _topk_body,
        grid=(G,),
        in_specs=[
            pl.BlockSpec((R, C), lambda i: (i, 0)),
            pl.BlockSpec((C, _N), lambda i: (0, 0)),
            pl.BlockSpec((C, _OUT), lambda i: (0, 0)),
            pl.BlockSpec((C, _OUT), lambda i: (0, 0)),
            pl.BlockSpec((1, _OUT), lambda i: (0, 0)),
        ],
        out_specs=[
            pl.BlockSpec((R, _K), lambda i: (i, 0)),
            pl.BlockSpec((R, _OUT), lambda i: (i, 0)),
            pl.BlockSpec((R, _OUT), lambda i: (i, 0)),
        ],
        out_shape=[
            jax.ShapeDtypeStruct((_N, _K), jnp.int32),
            jax.ShapeDtypeStruct((_N, _OUT), jnp.float32),
            jax.ShapeDtypeStruct((_N, _OUT), jnp.float32),
        ],
        scratch_shapes=[pltpu.VMEM((R, _N), jnp.float32)],
    )(x, x.T, Wu, Wa, b2)


_NW = 32
_ROWS_W = _N // _NW
_CH = 4
_NCH = _ROWS_W // _CH


def _sc_gather_relu_max(u, idx_flat, a):
    mesh = plsc.VectorSubcoreMesh(core_axis_name="c", subcore_axis_name="s")

    @functools.partial(
        pl.kernel,
        out_type=jax.ShapeDtypeStruct((_N, _OUT), jnp.float32),
        mesh=mesh,
        compiler_params=pltpu.CompilerParams(use_tc_tiling_on_sc=False),
        scratch_types=[
            pltpu.VMEM((_CH * _K,), jnp.int32),
            pltpu.VMEM((_CH * _K, _OUT), jnp.float32),
            pltpu.VMEM((_CH, _OUT), jnp.float32),
            pltpu.VMEM((_CH, _OUT), jnp.float32),
            pltpu.SemaphoreType.DMA,
        ],
    )
    def k(u_hbm, idx_hbm, a_hbm, out_hbm, idx_v, rows_v, a_v, o_v, sem):
        wid = lax.axis_index("s") * 2 + lax.axis_index("c")
        base = wid * _ROWS_W

        def chunk(ci, carry):
            rbase = base + ci * _CH
            pltpu.sync_copy(idx_hbm.at[pl.ds(rbase * _K, _CH * _K)], idx_v)
            pltpu.async_copy(u_hbm.at[idx_v], rows_v, sem).wait()
            pltpu.sync_copy(a_hbm.at[pl.ds(rbase, _CH), :], a_v)
            for r in range(_CH):
                def jstep(j, accs, r=r):
                    return tuple(
                        jnp.maximum(acc, rows_v[r * _K + j, pl.ds(c * 16, 16)])
                        for c, acc in enumerate(accs))
                accs = tuple(jnp.full((16,), _NEG, jnp.float32)
                             for _ in range(_OUT // 16))
                accs = lax.fori_loop(0, _K, jstep, accs)
                for c in range(_OUT // 16):
                    o_v[r, pl.ds(c * 16, 16)] = jnp.maximum(
                        a_v[r, pl.ds(c * 16, 16)] + accs[c], 0.0)
            pltpu.sync_copy(o_v, out_hbm.at[pl.ds(rbase, _CH), :])
            return carry

        lax.fori_loop(0, _NCH, chunk, 0)

    return k(u, idx_flat, a)


_HEAD_R = 512


def _head_body(x1_ref, x2_ref, x3_ref, wl_ref, bl_ref, wm1_ref, bm1_ref,
               wm2_ref, bm2_ref, wh_ref, bh_ref, o_ref):
    wl = wl_ref[...]
    h = (_dot(x1_ref[...], wl[0:_OUT]) + _dot(x2_ref[...], wl[_OUT:2 * _OUT])
         + _dot(x3_ref[...], wl[2 * _OUT:3 * _OUT]) + bl_ref[...])
    h = jnp.maximum(h, 0.0)
    h = jnp.maximum(_dot(h, wm1_ref[...]) + bm1_ref[...], 0.0)
    h = jnp.maximum(_dot(h, wm2_ref[...]) + bm2_ref[...], 0.0)
    o = _dot(h, wh_ref[...]) + bh_ref[...]
    shifted = o - jnp.max(o, axis=1, keepdims=True)
    o_ref[...] = shifted - jnp.log(
        jnp.sum(jnp.exp(shifted), axis=1, keepdims=True))


def _head(x1, x2, x3, Wl, bl, Wm1, bm1, Wm2, bm2, Wh, bh):
    R = _HEAD_R
    G = _N // R
    ncls = Wh.shape[1]
    full = lambda shp: pl.BlockSpec(shp, lambda i: tuple(0 for _ in shp))
    row = lambda shp: pl.BlockSpec(shp, lambda i: (i,) + (0,) * (len(shp) - 1))
    return pl.pallas_call(
        _head_body,
        grid=(G,),
        in_specs=[
            row((R, _OUT)), row((R, _OUT)), row((R, _OUT)),
            full(Wl.shape), full((1, bl.shape[0])),
            full(Wm1.shape), full((1, bm1.shape[0])),
            full(Wm2.shape), full((1, bm2.shape[0])),
            full(Wh.shape), full((1, bh.shape[0])),
        ],
        out_specs=row((R, ncls)),
        out_shape=jax.ShapeDtypeStruct((_N, ncls), jnp.float32),
    )(x1, x2, x3, Wl, bl.reshape(1, -1), Wm1, bm1.reshape(1, -1),
      Wm2, bm2.reshape(1, -1), Wh, bh.reshape(1, -1))


def _layer(xin, W, b):
    C = xin.shape[1]
    Wa = W[:C] - W[C:]
    Wu = W[C:]
    if C < 8:
        pad = 8 - C
        xin = jnp.concatenate([xin, jnp.zeros((_N, pad), xin.dtype)], axis=1)
        Wa = jnp.concatenate([Wa, jnp.zeros((pad, _OUT), Wa.dtype)], axis=0)
        Wu = jnp.concatenate([Wu, jnp.zeros((pad, _OUT), Wu.dtype)], axis=0)
    idx, u, a = _edge_topk(xin, Wu, Wa, b.reshape(1, _OUT))
    return _sc_gather_relu_max(u, idx.reshape(_N * _K), a)


def kernel(x, W1, b1, W2, b2, W3, b3, Wl, bl, Wm1, bm1, Wm2, bm2, Wh, bh):
    x1 = _layer(x, W1, b1)
    x2 = _layer(x1, W2, b2)
    x3 = _layer(x2, W3, b3)
    return _head(x1, x2, x3, Wl, bl, Wm1, bm1, Wm2, bm2, Wh, bh)
